# baseline (speedup 1.0000x reference)
import functools

import jax
import jax.numpy as jnp
from jax import lax
from jax.experimental import pallas as pl
from jax.experimental.pallas import tpu as pltpu
from jax.experimental.pallas import tpu_sc as plsc

NUM_HIDDENS = 768
LANES = 16
SEG_SLICES = NUM_HIDDENS // LANES
NC, NS = 2, 16
NW = NC * NS
ROWS = 4 * 8192
ROWS_PER_W = ROWS // NW
CHUNK = 64
CHUNKS_PER_W = ROWS_PER_W // CHUNK


def _sc_add(xf, seg2, flag):
    mesh = plsc.VectorSubcoreMesh(core_axis_name="c", subcore_axis_name="s")

    @functools.partial(
        pl.kernel,
        mesh=mesh,
        out_type=jax.ShapeDtypeStruct((ROWS, NUM_HIDDENS), jnp.float32),
        scratch_types=[
            pltpu.VMEM((2, NUM_HIDDENS), jnp.float32),
            pltpu.VMEM((LANES,), jnp.int32),
            pltpu.VMEM((NUM_HIDDENS,), jnp.float32),
            pltpu.VMEM((CHUNK, NUM_HIDDENS), jnp.float32),
        ],
    )
    def k(x_hbm, seg_hbm, flag_hbm, out_hbm, seg_v, flag_v, segsel_v, buf):
        wid = lax.axis_index("s") * NC + lax.axis_index("c")
        pltpu.sync_copy(seg_hbm, seg_v)
        pltpu.sync_copy(flag_hbm, flag_v)
        f = flag_v[...] != 0
        for j in range(SEG_SLICES):
            sl = pl.ds(j * LANES, LANES)
            segsel_v[sl] = jnp.where(f, seg_v[0, sl], seg_v[1, sl])
        row0 = wid * ROWS_PER_W

        def chunk_body(g, carry):
            base = row0 + g * CHUNK
            pltpu.sync_copy(x_hbm.at[pl.ds(base, CHUNK)], buf)

            def row_body(r, c):
                for j in range(SEG_SLICES):
                    sl = pl.ds(j * LANES, LANES)
                    plsc.addupdate(buf.at[r, sl], segsel_v[sl])
                return c

            lax.fori_loop(0, CHUNK, row_body, 0)
            pltpu.sync_copy(buf, out_hbm.at[pl.ds(base, CHUNK)])
            return carry

        lax.fori_loop(0, CHUNKS_PER_W, chunk_body, 0)

    return k(xf, seg2, flag)


def kernel(X, seg_emb, first_sentence):
    xf = X.reshape(ROWS, NUM_HIDDENS)
    seg2 = seg_emb.reshape(2, NUM_HIDDENS)
    flag = jnp.full((LANES,), first_sentence, dtype=jnp.int32)
    out = _sc_add(xf, seg2, flag)
    return out.reshape(X.shape)


# speedup vs baseline: 1.1469x; 1.1469x over previous
import functools

import jax
import jax.numpy as jnp
from jax import lax
from jax.experimental import pallas as pl
from jax.experimental.pallas import tpu as pltpu
from jax.experimental.pallas import tpu_sc as plsc

NUM_HIDDENS = 768
LANES = 16
SEG_SLICES = NUM_HIDDENS // LANES
NC, NS = 2, 16
NW = NC * NS
ROWS = 4 * 8192
ROWS_PER_W = ROWS // NW
CHUNK = 64
NCHUNKS = ROWS_PER_W // CHUNK


def _sc_add(xf, seg2, flag):
    mesh = plsc.VectorSubcoreMesh(core_axis_name="c", subcore_axis_name="s")

    @functools.partial(
        pl.kernel,
        mesh=mesh,
        out_type=jax.ShapeDtypeStruct((ROWS, NUM_HIDDENS), jnp.float32),
        scratch_types=[
            pltpu.VMEM((2, NUM_HIDDENS), jnp.float32),
            pltpu.VMEM((LANES,), jnp.int32),
            pltpu.VMEM((NUM_HIDDENS,), jnp.float32),
            pltpu.VMEM((CHUNK, NUM_HIDDENS), jnp.float32),
            pltpu.VMEM((CHUNK, NUM_HIDDENS), jnp.float32),
            pltpu.SemaphoreType.DMA,
            pltpu.SemaphoreType.DMA,
            pltpu.SemaphoreType.DMA,
            pltpu.SemaphoreType.DMA,
        ],
    )
    def k(x_hbm, seg_hbm, flag_hbm, out_hbm,
          seg_v, flag_v, segsel_v, b0, b1, si0, si1, so0, so1):
        wid = lax.axis_index("s") * NC + lax.axis_index("c")
        pltpu.sync_copy(seg_hbm, seg_v)
        pltpu.sync_copy(flag_hbm, flag_v)
        f = flag_v[...] != 0
        for j in range(SEG_SLICES):
            sl = pl.ds(j * LANES, LANES)
            segsel_v[sl] = jnp.where(f, seg_v[0, sl], seg_v[1, sl])
        row0 = wid * ROWS_PER_W

        bufs = (b0, b1)
        in_sems = (si0, si1)
        out_sems = (so0, so1)

        def in_copy(g):
            b = g % 2
            return pltpu.make_async_copy(
                x_hbm.at[pl.ds(row0 + g * CHUNK, CHUNK)], bufs[b], in_sems[b])

        def out_copy(g):
            b = g % 2
            return pltpu.make_async_copy(
                bufs[b], out_hbm.at[pl.ds(row0 + g * CHUNK, CHUNK)], out_sems[b])

        def compute(g):
            buf = bufs[g % 2]

            def row_body(r, c):
                for j in range(SEG_SLICES):
                    sl = pl.ds(j * LANES, LANES)
                    plsc.addupdate(buf.at[r, sl], segsel_v[sl])
                return c

            lax.fori_loop(0, CHUNK, row_body, 0)

        in_copy(0).start()
        for g in range(NCHUNKS):
            if g + 1 < NCHUNKS:
                if g >= 1:
                    out_copy(g - 1).wait()
                in_copy(g + 1).start()
            in_copy(g).wait()
            compute(g)
            out_copy(g).start()
        out_copy(NCHUNKS - 2).wait()
        out_copy(NCHUNKS - 1).wait()

    return k(xf, seg2, flag)


def kernel(X, seg_emb, first_sentence):
    xf = X.reshape(ROWS, NUM_HIDDENS)
    seg2 = seg_emb.reshape(2, NUM_HIDDENS)
    flag = jnp.full((LANES,), first_sentence, dtype=jnp.int32)
    out = _sc_add(xf, seg2, flag)
    return out.reshape(X.shape)


# speedup vs baseline: 2.6972x; 2.3518x over previous
import functools

import jax
import jax.numpy as jnp
from jax import lax
from jax.experimental import pallas as pl
from jax.experimental.pallas import tpu as pltpu
from jax.experimental.pallas import tpu_sc as plsc

NUM_HIDDENS = 768
LANES = 16
SEG_SLICES = NUM_HIDDENS // LANES
NC, NS = 2, 16
NW = NC * NS
ROWS = 4 * 8192
ROWS_PER_W = ROWS // NW
CHUNK = 64
NCHUNKS = ROWS_PER_W // CHUNK


def _sc_add(xf, seg2, flag):
    mesh = plsc.VectorSubcoreMesh(core_axis_name="c", subcore_axis_name="s")

    @functools.partial(
        pl.kernel,
        mesh=mesh,
        out_type=jax.ShapeDtypeStruct((ROWS, NUM_HIDDENS), jnp.float32),
        scratch_types=[
            pltpu.VMEM((2, NUM_HIDDENS), jnp.float32),
            pltpu.VMEM((LANES,), jnp.int32),
            pltpu.VMEM((CHUNK, NUM_HIDDENS), jnp.float32),
            pltpu.VMEM((CHUNK, NUM_HIDDENS), jnp.float32),
            pltpu.SemaphoreType.DMA,
            pltpu.SemaphoreType.DMA,
            pltpu.SemaphoreType.DMA,
            pltpu.SemaphoreType.DMA,
        ],
    )
    def k(x_hbm, seg_hbm, flag_hbm, out_hbm,
          seg_v, flag_v, b0, b1, si0, si1, so0, so1):
        wid = lax.axis_index("s") * NC + lax.axis_index("c")
        pltpu.sync_copy(seg_hbm, seg_v)
        pltpu.sync_copy(flag_hbm, flag_v)
        f = flag_v[...] != 0
        segs = [
            jnp.where(f, seg_v[0, pl.ds(j * LANES, LANES)],
                      seg_v[1, pl.ds(j * LANES, LANES)])
            for j in range(SEG_SLICES)
        ]
        row0 = wid * ROWS_PER_W

        bufs = (b0, b1)
        in_sems = (si0, si1)
        out_sems = (so0, so1)

        def in_copy(g):
            b = g % 2
            return pltpu.make_async_copy(
                x_hbm.at[pl.ds(row0 + g * CHUNK, CHUNK)], bufs[b], in_sems[b])

        def out_copy(g):
            b = g % 2
            return pltpu.make_async_copy(
                bufs[b], out_hbm.at[pl.ds(row0 + g * CHUNK, CHUNK)], out_sems[b])

        def compute(g):
            buf = bufs[g % 2]

            def row_body(r, c):
                for j in range(SEG_SLICES):
                    sl = pl.ds(j * LANES, LANES)
                    plsc.addupdate(buf.at[r, sl], segs[j])
                return c

            lax.fori_loop(0, CHUNK, row_body, 0)

        in_copy(0).start()
        for g in range(NCHUNKS):
            if g + 1 < NCHUNKS:
                if g >= 1:
                    out_copy(g - 1).wait()
                in_copy(g + 1).start()
            in_copy(g).wait()
            compute(g)
            out_copy(g).start()
        out_copy(NCHUNKS - 2).wait()
        out_copy(NCHUNKS - 1).wait()

    return k(xf, seg2, flag)


def kernel(X, seg_emb, first_sentence):
    xf = X.reshape(ROWS, NUM_HIDDENS)
    seg2 = seg_emb.reshape(2, NUM_HIDDENS)
    flag = jnp.full((LANES,), first_sentence, dtype=jnp.int32)
    out = _sc_add(xf, seg2, flag)
    return out.reshape(X.shape)
